# Initial kernel scaffold; baseline (speedup 1.0000x reference)
import jax
import jax.numpy as jnp
from jax.experimental import pallas as pl


def kernel(x, edge_index):
    raise NotImplementedError("write your pallas kernel here")


# speedup vs baseline: 7.5665x; 7.5665x over previous
import jax
import jax.numpy as jnp
from jax import lax
from jax.experimental import pallas as pl
from jax.experimental.pallas import tpu as pltpu
from jax.experimental.pallas import tpu_sc as plsc

N_NODES = 10000
N_EDGES = 320000
D = 128

NC = 2
NS = 16
NW = NC * NS

E_PER_W = N_EDGES // NW
CHUNK = 80
STEPS = E_PER_W // CHUNK


def _negate_body(x_ref, o_ref):
    o_ref[...] = -x_ref[...]


def _negate(x):
    return pl.pallas_call(
        _negate_body,
        out_shape=jax.ShapeDtypeStruct(x.shape, x.dtype),
        grid=(10,),
        in_specs=[pl.BlockSpec((N_NODES // 10, D), lambda i: (i, 0))],
        out_specs=pl.BlockSpec((N_NODES // 10, D), lambda i: (i, 0)),
    )(x)


def _sc_body(x_hbm, xneg_hbm, src_hbm, dst_hbm, out_hbm,
             idx_src, idx_dst, buf, sem):
    wid = lax.axis_index("s") * NC + lax.axis_index("c")
    base = wid * E_PER_W
    pltpu.sync_copy(src_hbm.at[pl.ds(base, E_PER_W)], idx_src)
    pltpu.sync_copy(dst_hbm.at[pl.ds(base, E_PER_W)], idx_dst)

    def step(c, carry):
        off = c * CHUNK
        pltpu.async_copy(
            x_hbm.at[idx_src.at[pl.ds(off, CHUNK)]], buf, sem).wait()
        pltpu.async_copy(
            xneg_hbm.at[idx_dst.at[pl.ds(off, CHUNK)]], buf, sem,
            add=True).wait()
        pltpu.sync_copy(buf, out_hbm.at[pl.ds(base + off, CHUNK)])
        return carry

    lax.fori_loop(0, STEPS, step, 0)


@jax.jit
def kernel(x, edge_index):
    xneg = _negate(x)
    src = edge_index[0]
    dst = edge_index[1]

    mesh = plsc.VectorSubcoreMesh(core_axis_name="c", subcore_axis_name="s")
    out = pl.kernel(
        _sc_body,
        out_type=jax.ShapeDtypeStruct((N_EDGES, D), jnp.float32),
        mesh=mesh,
        scratch_types=[
            pltpu.VMEM((E_PER_W,), jnp.int32),
            pltpu.VMEM((E_PER_W,), jnp.int32),
            pltpu.VMEM((CHUNK, D), jnp.float32),
            pltpu.SemaphoreType.DMA,
        ],
    )(x, xneg, src, dst)
    return out


# speedup vs baseline: 13.8469x; 1.8300x over previous
import jax
import jax.numpy as jnp
from jax import lax
from jax.experimental import pallas as pl
from jax.experimental.pallas import tpu as pltpu
from jax.experimental.pallas import tpu_sc as plsc

N_NODES = 10000
N_EDGES = 320000
D = 128

NC = 2
NS = 16
NW = NC * NS

E_PER_W = N_EDGES // NW
CHUNK = 80
STEPS = E_PER_W // CHUNK


def _negate_body(x_ref, o_ref):
    o_ref[...] = -x_ref[...]


def _negate(x):
    return pl.pallas_call(
        _negate_body,
        out_shape=jax.ShapeDtypeStruct(x.shape, x.dtype),
        grid=(10,),
        in_specs=[pl.BlockSpec((N_NODES // 10, D), lambda i: (i, 0))],
        out_specs=pl.BlockSpec((N_NODES // 10, D), lambda i: (i, 0)),
    )(x)


NBUF = 5
GROUPS = STEPS // NBUF


def _sc_body(x_hbm, xneg_hbm, src_hbm, dst_hbm, out_hbm,
             idx_src, idx_dst, bufs, sem_g, sem_a, sem_w):
    wid = lax.axis_index("s") * NC + lax.axis_index("c")
    base = wid * E_PER_W
    pltpu.sync_copy(src_hbm.at[pl.ds(base, E_PER_W)], idx_src)
    pltpu.sync_copy(dst_hbm.at[pl.ds(base, E_PER_W)], idx_dst)

    def do_group(g, first):
        g1 = []
        for b in range(NBUF):
            off = (g * NBUF + b) * CHUNK
            buf = bufs.at[pl.ds(b * CHUNK, CHUNK)]
            if not first:
                pltpu.make_async_copy(
                    buf, out_hbm.at[pl.ds(base + off, CHUNK)],
                    sem_w[b]).wait()
            g1.append(pltpu.async_copy(
                x_hbm.at[idx_src.at[pl.ds(off, CHUNK)]], buf, sem_g[b]))
        g2 = []
        for b in range(NBUF):
            off = (g * NBUF + b) * CHUNK
            buf = bufs.at[pl.ds(b * CHUNK, CHUNK)]
            g1[b].wait()
            g2.append(pltpu.async_copy(
                xneg_hbm.at[idx_dst.at[pl.ds(off, CHUNK)]], buf, sem_a[b],
                add=True))
        for b in range(NBUF):
            off = (g * NBUF + b) * CHUNK
            buf = bufs.at[pl.ds(b * CHUNK, CHUNK)]
            g2[b].wait()
            pltpu.async_copy(buf, out_hbm.at[pl.ds(base + off, CHUNK)],
                             sem_w[b])

    do_group(0, True)
    lax.fori_loop(1, GROUPS, lambda g, cr: (do_group(g, False), cr)[1], 0)
    for b in range(NBUF):
        buf = bufs.at[pl.ds(b * CHUNK, CHUNK)]
        pltpu.make_async_copy(
            buf, out_hbm.at[pl.ds(base + b * CHUNK, CHUNK)], sem_w[b]).wait()


@jax.jit
def kernel(x, edge_index):
    xneg = _negate(x)
    src = edge_index[0]
    dst = edge_index[1]

    mesh = plsc.VectorSubcoreMesh(core_axis_name="c", subcore_axis_name="s")
    out = pl.kernel(
        _sc_body,
        out_type=jax.ShapeDtypeStruct((N_EDGES, D), jnp.float32),
        mesh=mesh,
        scratch_types=[
            pltpu.VMEM((E_PER_W,), jnp.int32),
            pltpu.VMEM((E_PER_W,), jnp.int32),
            pltpu.VMEM((NBUF * CHUNK, D), jnp.float32),
            [pltpu.SemaphoreType.DMA] * NBUF,
            [pltpu.SemaphoreType.DMA] * NBUF,
            [pltpu.SemaphoreType.DMA] * NBUF,
        ],
    )(x, xneg, src, dst)
    return out


# speedup vs baseline: 14.4023x; 1.0401x over previous
import jax
import jax.numpy as jnp
from jax import lax
from jax.experimental import pallas as pl
from jax.experimental.pallas import tpu as pltpu
from jax.experimental.pallas import tpu_sc as plsc

N_NODES = 10000
N_EDGES = 320000
D = 128

NC = 2
NS = 16
NW = NC * NS

E_PER_W = N_EDGES // NW
CHUNK = 80
STEPS = E_PER_W // CHUNK
NBUF = 5
GROUPS = STEPS // NBUF
VPR = D // 16


def _sc_body(x_hbm, eidx_hbm, out_hbm,
             idx_src, idx_dst, bufa, bufb, sem_a, sem_b, sem_w):
    wid = lax.axis_index("s") * NC + lax.axis_index("c")
    base = wid * E_PER_W
    pltpu.sync_copy(eidx_hbm.at[pl.ds(base, E_PER_W)], idx_src)
    pltpu.sync_copy(eidx_hbm.at[pl.ds(N_EDGES + base, E_PER_W)], idx_dst)

    def do_group(g, first):
        ga, gb = [], []
        for b in range(NBUF):
            off = (g * NBUF + b) * CHUNK
            a = bufa.at[pl.ds(b * CHUNK, CHUNK)]
            bb = bufb.at[pl.ds(b * CHUNK, CHUNK)]
            if not first:
                pltpu.make_async_copy(
                    a, out_hbm.at[pl.ds(base + off, CHUNK)], sem_w[b]).wait()
            ga.append(pltpu.async_copy(
                x_hbm.at[idx_src.at[pl.ds(off, CHUNK)]], a, sem_a[b]))
            gb.append(pltpu.async_copy(
                x_hbm.at[idx_dst.at[pl.ds(off, CHUNK)]], bb, sem_b[b]))
        for b in range(NBUF):
            off = (g * NBUF + b) * CHUNK
            a = bufa.at[pl.ds(b * CHUNK, CHUNK)]
            ga[b].wait()
            gb[b].wait()

            def row(r, carry, b=b):
                rr = b * CHUNK + r
                for j in range(VPR):
                    s = pl.ds(j * 16, 16)
                    bufa[rr, s] = bufa[rr, s] - bufb[rr, s]
                return carry

            lax.fori_loop(0, CHUNK, row, 0)
            pltpu.async_copy(a, out_hbm.at[pl.ds(base + off, CHUNK)],
                             sem_w[b])

    do_group(0, True)
    lax.fori_loop(1, GROUPS, lambda g, cr: (do_group(g, False), cr)[1], 0)
    for b in range(NBUF):
        a = bufa.at[pl.ds(b * CHUNK, CHUNK)]
        pltpu.make_async_copy(
            a, out_hbm.at[pl.ds(base + b * CHUNK, CHUNK)], sem_w[b]).wait()


@jax.jit
def kernel(x, edge_index):
    eidx = edge_index.reshape(-1)

    mesh = plsc.VectorSubcoreMesh(core_axis_name="c", subcore_axis_name="s")
    out = pl.kernel(
        _sc_body,
        out_type=jax.ShapeDtypeStruct((N_EDGES, D), jnp.float32),
        mesh=mesh,
        scratch_types=[
            pltpu.VMEM((E_PER_W,), jnp.int32),
            pltpu.VMEM((E_PER_W,), jnp.int32),
            pltpu.VMEM((NBUF * CHUNK, D), jnp.float32),
            pltpu.VMEM((NBUF * CHUNK, D), jnp.float32),
            [pltpu.SemaphoreType.DMA] * NBUF,
            [pltpu.SemaphoreType.DMA] * NBUF,
            [pltpu.SemaphoreType.DMA] * NBUF,
        ],
    )(x, eidx)
    return out


# speedup vs baseline: 15.3148x; 1.0634x over previous
import jax
import jax.numpy as jnp
from jax import lax
from jax.experimental import pallas as pl
from jax.experimental.pallas import tpu as pltpu
from jax.experimental.pallas import tpu_sc as plsc

N_NODES = 10000
N_EDGES = 320000
D = 128

NC = 2
NS = 16
NW = NC * NS

E_PER_W = N_EDGES // NW
CHUNK = 80
STEPS = E_PER_W // CHUNK
NBUF = 3
GROUPS = (STEPS - 2) // NBUF

STG = 16
ROWS_T = 624
ROWS_LAST = N_NODES - 15 * ROWS_T


def _sc_body(x_hbm, eidx_hbm, out_hbm,
             idx_src, idx_dst, bufa, xneg_spm, sem_a, sem_w):
    cid = lax.axis_index("c")
    sid = lax.axis_index("s")
    wid = sid * NC + cid
    base = wid * E_PER_W

    rowbase = sid * ROWS_T
    nchunks = jnp.where(sid == NS - 1, ROWS_LAST // STG, ROWS_T // STG)
    tmp = bufa.at[pl.ds(0, STG)]

    def stage(c, carry):
        r = rowbase + c * STG
        pltpu.sync_copy(x_hbm.at[pl.ds(r, STG)], tmp)
        for row in range(STG):
            for jj in range(D // 16):
                s = pl.ds(jj * 16, 16)
                bufa[row, s] = -bufa[row, s]
        pltpu.sync_copy(tmp, xneg_spm.at[pl.ds(r, STG)])
        return carry

    lax.fori_loop(0, nchunks, stage, 0)
    plsc.subcore_barrier()

    pltpu.sync_copy(eidx_hbm.at[pl.ds(base, E_PER_W)], idx_src)
    pltpu.sync_copy(eidx_hbm.at[pl.ds(N_EDGES + base, E_PER_W)], idx_dst)

    def do_group(g, first):
        ga = []
        for b in range(NBUF):
            off = (g * NBUF + b) * CHUNK
            a = bufa.at[pl.ds(b * CHUNK, CHUNK)]
            if not first:
                pltpu.make_async_copy(
                    a, out_hbm.at[pl.ds(base + off, CHUNK)], sem_w[b]).wait()
            ga.append(pltpu.async_copy(
                x_hbm.at[idx_src.at[pl.ds(off, CHUNK)]], a, sem_a[b]))
        for b in range(NBUF):
            off = (g * NBUF + b) * CHUNK
            a = bufa.at[pl.ds(b * CHUNK, CHUNK)]
            ga[b].wait()
            pltpu.async_copy(
                xneg_spm.at[idx_dst.at[pl.ds(off, CHUNK)]], a, sem_a[b],
                add=True).wait()
            pltpu.async_copy(a, out_hbm.at[pl.ds(base + off, CHUNK)],
                             sem_w[b])

    do_group(0, True)
    lax.fori_loop(1, GROUPS, lambda g, cr: (do_group(g, False), cr)[1], 0)

    for b in range(STEPS - NBUF * GROUPS):
        c = NBUF * GROUPS + b
        off = c * CHUNK
        a = bufa.at[pl.ds(b * CHUNK, CHUNK)]
        pltpu.make_async_copy(
            a, out_hbm.at[pl.ds(base + off, CHUNK)], sem_w[b]).wait()
        pltpu.async_copy(
            x_hbm.at[idx_src.at[pl.ds(off, CHUNK)]], a, sem_a[b]).wait()
        pltpu.async_copy(
            xneg_spm.at[idx_dst.at[pl.ds(off, CHUNK)]], a, sem_a[b],
            add=True).wait()
        pltpu.async_copy(a, out_hbm.at[pl.ds(base + off, CHUNK)], sem_w[b])

    for b in range(NBUF):
        a = bufa.at[pl.ds(b * CHUNK, CHUNK)]
        pltpu.make_async_copy(
            a, out_hbm.at[pl.ds(base + b * CHUNK, CHUNK)], sem_w[b]).wait()


@jax.jit
def kernel(x, edge_index):
    eidx = edge_index.reshape(-1)

    mesh = plsc.VectorSubcoreMesh(core_axis_name="c", subcore_axis_name="s")
    out = pl.kernel(
        _sc_body,
        out_type=jax.ShapeDtypeStruct((N_EDGES, D), jnp.float32),
        mesh=mesh,
        scratch_types=[
            pltpu.VMEM((E_PER_W,), jnp.int32),
            pltpu.VMEM((E_PER_W,), jnp.int32),
            pltpu.VMEM((NBUF * CHUNK, D), jnp.float32),
            pltpu.VMEM_SHARED((N_NODES, D), jnp.float32),
            [pltpu.SemaphoreType.DMA] * NBUF,
            [pltpu.SemaphoreType.DMA] * NBUF,
        ],
    )(x, eidx)
    return out
